# Initial kernel scaffold; baseline (speedup 1.0000x reference)
import functools

import jax
import jax.numpy as jnp
from jax import lax
from jax.experimental import pallas as pl
from jax.experimental.pallas import tpu as pltpu
from jax.experimental.pallas import tpu_sc as plsc

NC, NS, L = 2, 16, 16
NW = NC * NS
N = 100000
PTS_W = 3136
NPAD = NW * PTS_W
NCH = PTS_W // L
NPC = 50000
PCW = 3136
NPCPAD = NS * PCW
GRID = 64 * 64 * 64
OUTC = 963

_LD = (64, 32, 16, 8)
_LC = (64, 128, 256, 500)
_LOFF = (3, 67, 195, 451)
_LSCALE = (1.0, 0.5, 0.25, 0.125)

_mesh = plsc.VectorSubcoreMesh(
    core_axis_name="c", subcore_axis_name="s", num_cores=NC, num_subcores=NS)


@functools.partial(
    pl.kernel,
    out_type=jax.ShapeDtypeStruct((GRID,), jnp.int32),
    mesh=_mesh,
    scratch_types=[
        pltpu.VMEM((PCW,), jnp.float32),
        pltpu.VMEM((PCW,), jnp.float32),
        pltpu.VMEM((PCW,), jnp.float32),
        pltpu.VMEM((28, 112), jnp.int32),
        pltpu.VMEM((28, 112), jnp.float32),
        pltpu.VMEM_SHARED((3 * GRID,), jnp.float32),
        pltpu.VMEM((3, 8192), jnp.float32),
        pltpu.VMEM((8192,), jnp.int32),
    ],
)
def _occ_build(pc_hbm, occ_hbm, px, py, pz, idxb, valb, grids, gbuf, obuf):
    cid = lax.axis_index("c")
    sid = lax.axis_index("s")
    wid = sid * NC + cid
    lanes = lax.iota(jnp.int32, L)
    zeros16 = jnp.zeros((L,), jnp.float32)

    def _zb(i, _):
        gbuf[0, pl.ds(i * L, L)] = zeros16
        return 0
    lax.fori_loop(0, 8192 // L, _zb, 0)
    for j in range(6):
        pltpu.sync_copy(gbuf.at[0],
                        grids.at[pl.ds(sid * (3 * GRID // NS) + j * 8192, 8192)])
    plsc.subcore_barrier()

    for g in range(3):
        pltpu.sync_copy(pc_hbm.at[g, 0, pl.ds(sid * PCW, PCW)], px)
        pltpu.sync_copy(pc_hbm.at[g, 1, pl.ds(sid * PCW, PCW)], py)
        pltpu.sync_copy(pc_hbm.at[g, 2, pl.ds(sid * PCW, PCW)], pz)

        def _chunk(t, _):
            j = t // 7
            o = (t % 7) * L
            off = j * 112 + o
            x = px[pl.ds(off, L)]
            y = py[pl.ds(off, L)]
            z = pz[pl.ds(off, L)]
            h1 = jnp.minimum(jnp.maximum(32.0 * y + 32.0, 0.0), 63.0).astype(jnp.int32)
            w1 = jnp.minimum(jnp.maximum(32.0 * x + 32.0, 0.0), 63.0).astype(jnp.int32)
            c1 = jnp.minimum(jnp.maximum(32.0 * z + 32.0, 0.0), 63.0).astype(jnp.int32)
            flat = (h1 * 64 + w1) * 64 + c1 + g * GRID
            gidx = sid * PCW + off + lanes
            val = jnp.where(gidx < NPC, 1.0, 0.0).astype(jnp.float32)
            idxb[j, pl.ds(o, L)] = flat
            valb[j, pl.ds(o, L)] = val
            return 0
        lax.fori_loop(0, NCH, _chunk, 0)
        for j in range(28):
            pltpu.sync_copy(valb.at[j], grids.at[idxb.at[j]], add=True)
    plsc.subcore_barrier()

    base = wid * (GRID // NW)
    for g in range(3):
        pltpu.sync_copy(grids.at[pl.ds(g * GRID + base, 8192)], gbuf.at[g])

    def _pk(i, _):
        s = i * L
        b0 = jnp.where(gbuf[0, pl.ds(s, L)] > 0.5, 1, 0).astype(jnp.int32)
        b1 = jnp.where(gbuf[1, pl.ds(s, L)] > 0.5, 2, 0).astype(jnp.int32)
        b2 = jnp.where(gbuf[2, pl.ds(s, L)] > 0.5, 4, 0).astype(jnp.int32)
        obuf[pl.ds(s, L)] = b0 | b1 | b2
        return 0
    lax.fori_loop(0, 8192 // L, _pk, 0)
    pltpu.sync_copy(obuf, occ_hbm.at[pl.ds(base, 8192)])


@functools.partial(
    pl.kernel,
    out_type=jax.ShapeDtypeStruct((N, OUTC), jnp.float32),
    mesh=_mesh,
    scratch_types=[
        pltpu.VMEM((3, PTS_W), jnp.float32),
        pltpu.VMEM((64,), jnp.int32),
        pltpu.VMEM((64,), jnp.int32),
        pltpu.VMEM((64,), jnp.int32),
        pltpu.VMEM((64,), jnp.int32),
        pltpu.VMEM((64, 64), jnp.float32),
        pltpu.VMEM((64, 128), jnp.float32),
        pltpu.VMEM((64, 256), jnp.float32),
        pltpu.VMEM((64, 500), jnp.float32),
        pltpu.VMEM((64,), jnp.int32),
        pltpu.VMEM((16, 16), jnp.float32),
        pltpu.VMEM((16, OUTC), jnp.float32),
        pltpu.SemaphoreType.DMA,
    ],
)
def _project(coords_hbm, f0, f1, f2, f3, occ_hbm, out_hbm,
             cbuf, idx0, idx1, idx2, idx3, rows0, rows1, rows2, rows3,
             occv, wbuf, stage, gsem):
    cid = lax.axis_index("c")
    sid = lax.axis_index("s")
    wid = sid * NC + cid
    base = wid * PTS_W
    for a in range(3):
        pltpu.sync_copy(coords_hbm.at[a, pl.ds(base, PTS_W)], cbuf.at[a])
    nch = jnp.minimum(PTS_W, N - base) // L
    lanes = lax.iota(jnp.int32, L)
    idxs = (idx0, idx1, idx2, idx3)
    rows = (rows0, rows1, rows2, rows3)
    tables = (f0, f1, f2, f3)

    def _chunk(gi, _):
        off = gi * L
        x = cbuf[0, pl.ds(off, L)]
        y = cbuf[1, pl.ds(off, L)]
        z = cbuf[2, pl.ds(off, L)]
        h = jnp.minimum(jnp.maximum(32.0 * y + 32.0, 0.0), 63.0)
        w = jnp.minimum(jnp.maximum(32.0 * x + 32.0, 0.0), 63.0)
        c = jnp.minimum(jnp.maximum(32.0 * z + 32.0, 0.0), 63.0)
        w0vecs = None
        for lvl in range(4):
            d = _LD[lvl]
            hx = h * _LSCALE[lvl]
            wx = w * _LSCALE[lvl]
            cx = c * _LSCALE[lvl]
            xi1 = hx.astype(jnp.int32)
            x1f = xi1.astype(jnp.float32)
            xi2 = xi1 + (hx > x1f).astype(jnp.int32)
            x2f = xi2.astype(jnp.float32)
            yi1 = wx.astype(jnp.int32)
            y1f = yi1.astype(jnp.float32)
            yi2 = yi1 + (wx > y1f).astype(jnp.int32)
            y2f = yi2.astype(jnp.float32)
            zt = cx.astype(jnp.int32)
            zi1 = zt + (cx > zt.astype(jnp.float32)).astype(jnp.int32)
            i11 = (xi1 * d + yi1) * d + zi1
            i21 = (xi2 * d + yi1) * d + zi1
            i12 = (xi1 * d + yi2) * d + zi1
            i22 = (xi2 * d + yi2) * d + zi1
            dx2 = x2f - hx
            dx1 = hx - x1f
            dy2 = y2f - wx
            dy1 = wx - y1f
            w11 = dx2 * dy2
            w21 = dx1 * dy2
            w12 = dx2 * dy1
            w22 = dx1 * dy1
            ib = idxs[lvl]
            ib[pl.ds(0, L)] = i11
            ib[pl.ds(16, L)] = i21
            ib[pl.ds(32, L)] = i12
            ib[pl.ds(48, L)] = i22
            wbuf[4 * lvl + 0, :] = w11
            wbuf[4 * lvl + 1, :] = w21
            wbuf[4 * lvl + 2, :] = w12
            wbuf[4 * lvl + 3, :] = w22
            if lvl == 0:
                w0vecs = (w11, w21, w12, w22)
        dmas = [pltpu.async_copy(tables[lvl].at[idxs[lvl]], rows[lvl], gsem)
                for lvl in range(4)]
        dmas.append(pltpu.async_copy(occ_hbm.at[idx0], occv, gsem))
        for dsc in dmas:
            dsc.wait()

        plsc.store_scatter(stage, [lanes, jnp.full((L,), 0, jnp.int32)], x)
        plsc.store_scatter(stage, [lanes, jnp.full((L,), 1, jnp.int32)], y)
        plsc.store_scatter(stage, [lanes, jnp.full((L,), 2, jnp.int32)], z)

        acc = [jnp.zeros((L,), jnp.float32) for _ in range(3)]
        for k in range(4):
            word = occv[pl.ds(k * L, L)]
            wk = w0vecs[k]
            for g in range(3):
                bit = ((word >> g) & 1).astype(jnp.float32)
                acc[g] = acc[g] + wk * bit
        for g in range(3):
            for j in range(4):
                col = 951 + 4 * g + j
                plsc.store_scatter(
                    stage, [lanes, jnp.full((L,), col, jnp.int32)], acc[g])

        def _pbody(p, _):
            for lvl in range(4):
                cdim = _LC[lvl]
                coff = _LOFF[lvl]
                rb = rows[lvl]
                wv = [jnp.full((L,), wbuf[4 * lvl + k, p], jnp.float32)
                      for k in range(4)]

                def _cc(ci, _c, rb=rb, wv=wv, coff=coff, p=p):
                    s = ci * L
                    q11 = rb[p, pl.ds(s, L)]
                    q21 = rb[16 + p, pl.ds(s, L)]
                    q12 = rb[32 + p, pl.ds(s, L)]
                    q22 = rb[48 + p, pl.ds(s, L)]
                    stage[p, pl.ds(coff + s, L)] = (
                        wv[0] * q11 + wv[1] * q21 + wv[2] * q12 + wv[3] * q22)
                    return 0
                lax.fori_loop(0, cdim // L, _cc, 0)
                if cdim % L:
                    s = cdim - L
                    q11 = rb[p, pl.ds(s, L)]
                    q21 = rb[16 + p, pl.ds(s, L)]
                    q12 = rb[32 + p, pl.ds(s, L)]
                    q22 = rb[48 + p, pl.ds(s, L)]
                    stage[p, pl.ds(coff + s, L)] = (
                        wv[0] * q11 + wv[1] * q21 + wv[2] * q12 + wv[3] * q22)
            return 0
        lax.fori_loop(0, L, _pbody, 0)
        pltpu.sync_copy(stage, out_hbm.at[pl.ds(base + off, L)])
        return 0
    lax.fori_loop(0, nch, _chunk, 0)


def kernel(inputs, img_feat0, img_feat1, img_feat2, img_feat3,
           pc_feat0, pc_feat1, pc_feat2):
    coords = jnp.pad(inputs, ((0, NPAD - N), (0, 0)), constant_values=-1.0).T
    pc = jnp.stack([pc_feat0, pc_feat1, pc_feat2], axis=0)
    pc = jnp.pad(pc, ((0, 0), (0, NPCPAD - NPC), (0, 0)))
    pc = pc.transpose(0, 2, 1)
    f0 = img_feat0.reshape(GRID, 64)
    f1 = img_feat1.reshape(32 * 32 * 32, 128)
    f2 = img_feat2.reshape(16 * 16 * 16, 256)
    f3 = img_feat3.reshape(8 * 8 * 8, 500)
    occ = _occ_build(pc)
    return _project(coords, f0, f1, f2, f3, occ)


# speedup vs baseline: 9.5797x; 9.5797x over previous
import functools

import jax
import jax.numpy as jnp
from jax import lax
from jax.experimental import pallas as pl
from jax.experimental.pallas import tpu as pltpu
from jax.experimental.pallas import tpu_sc as plsc

NC, NS, L = 2, 16, 16
NW = NC * NS
N = 100000
PTS_W = 3136
NPAD = NW * PTS_W
NCH = PTS_W // L
NPC = 50000
PCW = 3136
NPCPAD = NS * PCW
GRID = 64 * 64 * 64
OUTC = 963

_LD = (64, 32, 16, 8)
_LC = (64, 128, 256, 500)
_LCP = (64, 128, 256, 512)
_LOFF = (3, 67, 195, 451)
_LSCALE = (1.0, 0.5, 0.25, 0.125)

_mesh = plsc.VectorSubcoreMesh(
    core_axis_name="c", subcore_axis_name="s", num_cores=NC, num_subcores=NS)


@functools.partial(
    pl.kernel,
    out_type=jax.ShapeDtypeStruct((GRID,), jnp.float32),
    mesh=_mesh,
    compiler_params=pltpu.CompilerParams(needs_layout_passes=False),
    scratch_types=[
        pltpu.VMEM((PCW,), jnp.float32),
        pltpu.VMEM((PCW,), jnp.float32),
        pltpu.VMEM((PCW,), jnp.float32),
        pltpu.VMEM((112,), jnp.int32),
        pltpu.VMEM((112,), jnp.float32),
        pltpu.VMEM_SHARED((3 * GRID,), jnp.float32),
        pltpu.VMEM((3 * 8192,), jnp.float32),
        pltpu.VMEM((8192,), jnp.float32),
    ],
)
def _occ_build(pc_hbm, occ_hbm, px, py, pz, idxb, valb, grids, gbuf, obuf):
    cid = lax.axis_index("c")
    sid = lax.axis_index("s")
    wid = sid * NC + cid
    lanes = lax.iota(jnp.int32, L)
    zeros16 = jnp.zeros((L,), jnp.float32)

    def _zb(i, _):
        gbuf[pl.ds(i * L, L)] = zeros16
        return 0
    lax.fori_loop(0, 8192 // L, _zb, 0)
    for j in range(6):
        pltpu.sync_copy(gbuf.at[pl.ds(0, 8192)],
                        grids.at[pl.ds(sid * (3 * GRID // NS) + j * 8192, 8192)])
    plsc.subcore_barrier()

    for g in range(3):
        gb = 3 * g * NPCPAD
        pltpu.sync_copy(pc_hbm.at[pl.ds(gb + sid * PCW, PCW)], px)
        pltpu.sync_copy(pc_hbm.at[pl.ds(gb + NPCPAD + sid * PCW, PCW)], py)
        pltpu.sync_copy(pc_hbm.at[pl.ds(gb + 2 * NPCPAD + sid * PCW, PCW)], pz)

        def _grp(j, _, g=g):
            for t in range(7):
                o = t * L
                off = j * 112 + o
                x = px[pl.ds(off, L)]
                y = py[pl.ds(off, L)]
                z = pz[pl.ds(off, L)]
                h1 = jnp.minimum(jnp.maximum(32.0 * y + 32.0, 0.0),
                                 63.0).astype(jnp.int32)
                w1 = jnp.minimum(jnp.maximum(32.0 * x + 32.0, 0.0),
                                 63.0).astype(jnp.int32)
                c1 = jnp.minimum(jnp.maximum(32.0 * z + 32.0, 0.0),
                                 63.0).astype(jnp.int32)
                flat = (h1 * 64 + w1) * 64 + c1 + g * GRID
                gidx = sid * PCW + off + lanes
                val = jnp.where(gidx < NPC, 1.0, 0.0).astype(jnp.float32)
                idxb[pl.ds(o, L)] = flat
                valb[pl.ds(o, L)] = val
            pltpu.sync_copy(valb, grids.at[idxb], add=True)
            return 0
        lax.fori_loop(0, PCW // 112, _grp, 0)
    plsc.subcore_barrier()

    base = wid * (GRID // NW)
    for g in range(3):
        pltpu.sync_copy(grids.at[pl.ds(g * GRID + base, 8192)],
                        gbuf.at[pl.ds(g * 8192, 8192)])

    def _pk(i, _):
        s = i * L
        b0 = jnp.where(gbuf[pl.ds(s, L)] > 0.5, 1.0, 0.0)
        b1 = jnp.where(gbuf[pl.ds(8192 + s, L)] > 0.5, 2.0, 0.0)
        b2 = jnp.where(gbuf[pl.ds(16384 + s, L)] > 0.5, 4.0, 0.0)
        obuf[pl.ds(s, L)] = (b0 + b1 + b2).astype(jnp.float32)
        return 0
    lax.fori_loop(0, 8192 // L, _pk, 0)
    pltpu.sync_copy(obuf, occ_hbm.at[pl.ds(base, 8192)])


@functools.partial(
    pl.kernel,
    out_type=jax.ShapeDtypeStruct((N, OUTC), jnp.float32),
    mesh=_mesh,
    compiler_params=pltpu.CompilerParams(needs_layout_passes=False),
    scratch_types=[
        pltpu.VMEM((3 * PTS_W,), jnp.float32),
        pltpu.VMEM((64,), jnp.int32),
        pltpu.VMEM((64,), jnp.int32),
        pltpu.VMEM((64,), jnp.int32),
        pltpu.VMEM((64,), jnp.int32),
        pltpu.VMEM((64, 128), jnp.float32),
        pltpu.VMEM((64, 128), jnp.float32),
        pltpu.VMEM((64, 256), jnp.float32),
        pltpu.VMEM((64, 512), jnp.float32),
        pltpu.VMEM((16, 16), jnp.float32),
        pltpu.VMEM((16, OUTC), jnp.float32),
        pltpu.SemaphoreType.DMA,
    ],
)
def _project(coords_hbm, f0, f1, f2, f3, out_hbm,
             cbuf, idx0, idx1, idx2, idx3, rows0, rows1, rows2, rows3,
             wbuf, stage, gsem):
    cid = lax.axis_index("c")
    sid = lax.axis_index("s")
    wid = sid * NC + cid
    base = wid * PTS_W
    for a in range(3):
        pltpu.sync_copy(coords_hbm.at[pl.ds(a * NPAD + base, PTS_W)],
                        cbuf.at[pl.ds(a * PTS_W, PTS_W)])
    nch = jnp.minimum(PTS_W, N - base) // L
    lanes = lax.iota(jnp.int32, L)
    idxs = (idx0, idx1, idx2, idx3)
    rows = (rows0, rows1, rows2, rows3)
    tables = (f0, f1, f2, f3)

    def _chunk(gi, _):
        off = gi * L
        x = cbuf[pl.ds(off, L)]
        y = cbuf[pl.ds(PTS_W + off, L)]
        z = cbuf[pl.ds(2 * PTS_W + off, L)]
        h = jnp.minimum(jnp.maximum(32.0 * y + 32.0, 0.0), 63.0)
        w = jnp.minimum(jnp.maximum(32.0 * x + 32.0, 0.0), 63.0)
        c = jnp.minimum(jnp.maximum(32.0 * z + 32.0, 0.0), 63.0)
        w0vecs = None
        for lvl in range(4):
            d = _LD[lvl]
            hx = h * _LSCALE[lvl]
            wx = w * _LSCALE[lvl]
            cx = c * _LSCALE[lvl]
            xi1 = hx.astype(jnp.int32)
            x1f = xi1.astype(jnp.float32)
            xi2 = xi1 + jnp.where(hx > x1f, 1, 0).astype(jnp.int32)
            x2f = xi2.astype(jnp.float32)
            yi1 = wx.astype(jnp.int32)
            y1f = yi1.astype(jnp.float32)
            yi2 = yi1 + jnp.where(wx > y1f, 1, 0).astype(jnp.int32)
            y2f = yi2.astype(jnp.float32)
            zt = cx.astype(jnp.int32)
            zi1 = zt + jnp.where(cx > zt.astype(jnp.float32), 1, 0).astype(jnp.int32)
            i11 = (xi1 * d + yi1) * d + zi1
            i21 = (xi2 * d + yi1) * d + zi1
            i12 = (xi1 * d + yi2) * d + zi1
            i22 = (xi2 * d + yi2) * d + zi1
            dx2 = x2f - hx
            dx1 = hx - x1f
            dy2 = y2f - wx
            dy1 = wx - y1f
            w11 = dx2 * dy2
            w21 = dx1 * dy2
            w12 = dx2 * dy1
            w22 = dx1 * dy1
            ib = idxs[lvl]
            ib[pl.ds(0, L)] = i11
            ib[pl.ds(16, L)] = i21
            ib[pl.ds(32, L)] = i12
            ib[pl.ds(48, L)] = i22
            wbuf[4 * lvl + 0, :] = w11
            wbuf[4 * lvl + 1, :] = w21
            wbuf[4 * lvl + 2, :] = w12
            wbuf[4 * lvl + 3, :] = w22
            if lvl == 0:
                w0vecs = (w11, w21, w12, w22)
        dmas = [pltpu.async_copy(tables[lvl].at[idxs[lvl]], rows[lvl], gsem)
                for lvl in range(4)]
        for dsc in dmas:
            dsc.wait()

        def _pbody(p, _):
            pfull = jnp.full((L,), p, jnp.int32)
            for lvl in range(4):
                cdim = _LCP[lvl]
                coff = _LOFF[lvl]
                rb = rows[lvl]
                wv = [plsc.load_gather(
                          wbuf,
                          [jnp.full((L,), 4 * lvl + k, jnp.int32), pfull])
                      for k in range(4)]

                def _cc(ci, _c, rb=rb, wv=wv, coff=coff, p=p,
                        pfull=pfull):
                    s = ci * L
                    q11 = rb[p, pl.ds(s, L)]
                    q21 = rb[16 + p, pl.ds(s, L)]
                    q12 = rb[32 + p, pl.ds(s, L)]
                    q22 = rb[48 + p, pl.ds(s, L)]
                    plsc.store_scatter(
                        stage, [pfull, coff + s + lanes],
                        wv[0] * q11 + wv[1] * q21 + wv[2] * q12 + wv[3] * q22)
                    return 0
                lax.fori_loop(0, cdim // L, _cc, 0)
            return 0
        lax.fori_loop(0, L, _pbody, 0)

        plsc.store_scatter(stage, [lanes, jnp.full((L,), 0, jnp.int32)], x)
        plsc.store_scatter(stage, [lanes, jnp.full((L,), 1, jnp.int32)], y)
        plsc.store_scatter(stage, [lanes, jnp.full((L,), 2, jnp.int32)], z)

        acc = [jnp.zeros((L,), jnp.float32) for _ in range(3)]
        col64 = jnp.full((L,), 64, jnp.int32)
        for k in range(4):
            v = plsc.load_gather(rows0, [k * L + lanes, col64])
            wk = w0vecs[k]
            b2 = jnp.where(v >= 4.0, 1.0, 0.0)
            v = v - 4.0 * b2
            b1 = jnp.where(v >= 2.0, 1.0, 0.0)
            b0 = v - 2.0 * b1
            for g, bit in enumerate((b0, b1, b2)):
                acc[g] = acc[g] + wk * bit
        for g in range(3):
            for j in range(4):
                col = 951 + 4 * g + j
                plsc.store_scatter(
                    stage, [lanes, jnp.full((L,), col, jnp.int32)], acc[g])
        pltpu.sync_copy(stage, out_hbm.at[pl.ds(base + off, L)])
        return 0
    lax.fori_loop(0, nch, _chunk, 0)


def kernel(inputs, img_feat0, img_feat1, img_feat2, img_feat3,
           pc_feat0, pc_feat1, pc_feat2):
    coords = jnp.pad(inputs, ((0, NPAD - N), (0, 0)),
                     constant_values=-1.0).T.reshape(-1)
    pc = jnp.stack([pc_feat0, pc_feat1, pc_feat2], axis=0)
    pc = jnp.pad(pc, ((0, 0), (0, NPCPAD - NPC), (0, 0)))
    pc = pc.transpose(0, 2, 1).reshape(-1)
    f0 = img_feat0.reshape(GRID, 64)
    f1 = img_feat1.reshape(32 * 32 * 32, 128)
    f2 = img_feat2.reshape(16 * 16 * 16, 256)
    f3 = jnp.pad(img_feat3.reshape(8 * 8 * 8, 500), ((0, 0), (0, 12)))
    occf = _occ_build(pc)
    f0c = jnp.concatenate(
        [f0, occf[:, None], jnp.zeros((GRID, 63), jnp.float32)], axis=1)
    return _project(coords, f0c, f1, f2, f3)


# speedup vs baseline: 9.8426x; 1.0274x over previous
import functools

import jax
import jax.numpy as jnp
from jax import lax
from jax.experimental import pallas as pl
from jax.experimental.pallas import tpu as pltpu
from jax.experimental.pallas import tpu_sc as plsc

NC, NS, L = 2, 16, 16
NW = NC * NS
N = 100000
PTS_W = 3136
NPAD = NW * PTS_W
NCH = PTS_W // L
NPC = 50000
PCW = 3136
NPCPAD = NS * PCW
GRID = 64 * 64 * 64
OUTC = 963

_LD = (64, 32, 16, 8)
_LC = (64, 128, 256, 500)
_LCP = (64, 128, 256, 512)
_LOFF = (3, 67, 195, 451)
_LSCALE = (1.0, 0.5, 0.25, 0.125)

_mesh = plsc.VectorSubcoreMesh(
    core_axis_name="c", subcore_axis_name="s", num_cores=NC, num_subcores=NS)


@functools.partial(
    pl.kernel,
    out_type=jax.ShapeDtypeStruct((GRID,), jnp.float32),
    mesh=_mesh,
    compiler_params=pltpu.CompilerParams(needs_layout_passes=False),
    scratch_types=[
        pltpu.VMEM((PCW,), jnp.float32),
        pltpu.VMEM((PCW,), jnp.float32),
        pltpu.VMEM((PCW,), jnp.float32),
        pltpu.VMEM((112,), jnp.int32),
        pltpu.VMEM((112,), jnp.float32),
        pltpu.VMEM_SHARED((3 * GRID,), jnp.float32),
        pltpu.VMEM((3 * 8192,), jnp.float32),
        pltpu.VMEM((8192,), jnp.float32),
    ],
)
def _occ_build(pc_hbm, occ_hbm, px, py, pz, idxb, valb, grids, gbuf, obuf):
    cid = lax.axis_index("c")
    sid = lax.axis_index("s")
    wid = sid * NC + cid
    lanes = lax.iota(jnp.int32, L)
    zeros16 = jnp.zeros((L,), jnp.float32)

    def _zb(i, _):
        gbuf[pl.ds(i * L, L)] = zeros16
        return 0
    lax.fori_loop(0, 8192 // L, _zb, 0)
    for j in range(6):
        pltpu.sync_copy(gbuf.at[pl.ds(0, 8192)],
                        grids.at[pl.ds(sid * (3 * GRID // NS) + j * 8192, 8192)])
    plsc.subcore_barrier()

    for g in range(3):
        gb = 3 * g * NPCPAD
        pltpu.sync_copy(pc_hbm.at[pl.ds(gb + sid * PCW, PCW)], px)
        pltpu.sync_copy(pc_hbm.at[pl.ds(gb + NPCPAD + sid * PCW, PCW)], py)
        pltpu.sync_copy(pc_hbm.at[pl.ds(gb + 2 * NPCPAD + sid * PCW, PCW)], pz)

        def _grp(j, _, g=g):
            for t in range(7):
                o = t * L
                off = j * 112 + o
                x = px[pl.ds(off, L)]
                y = py[pl.ds(off, L)]
                z = pz[pl.ds(off, L)]
                h1 = jnp.minimum(jnp.maximum(32.0 * y + 32.0, 0.0),
                                 63.0).astype(jnp.int32)
                w1 = jnp.minimum(jnp.maximum(32.0 * x + 32.0, 0.0),
                                 63.0).astype(jnp.int32)
                c1 = jnp.minimum(jnp.maximum(32.0 * z + 32.0, 0.0),
                                 63.0).astype(jnp.int32)
                flat = (h1 * 64 + w1) * 64 + c1 + g * GRID
                gidx = sid * PCW + off + lanes
                val = jnp.where(gidx < NPC, 1.0, 0.0).astype(jnp.float32)
                idxb[pl.ds(o, L)] = flat
                valb[pl.ds(o, L)] = val
            pltpu.sync_copy(valb, grids.at[idxb], add=True)
            return 0
        lax.fori_loop(0, PCW // 112, _grp, 0)
    plsc.subcore_barrier()

    base = wid * (GRID // NW)
    for g in range(3):
        pltpu.sync_copy(grids.at[pl.ds(g * GRID + base, 8192)],
                        gbuf.at[pl.ds(g * 8192, 8192)])

    def _pk(i, _):
        s = i * L
        b0 = jnp.where(gbuf[pl.ds(s, L)] > 0.5, 1.0, 0.0)
        b1 = jnp.where(gbuf[pl.ds(8192 + s, L)] > 0.5, 2.0, 0.0)
        b2 = jnp.where(gbuf[pl.ds(16384 + s, L)] > 0.5, 4.0, 0.0)
        obuf[pl.ds(s, L)] = (b0 + b1 + b2).astype(jnp.float32)
        return 0
    lax.fori_loop(0, 8192 // L, _pk, 0)
    pltpu.sync_copy(obuf, occ_hbm.at[pl.ds(base, 8192)])


@functools.partial(
    pl.kernel,
    out_type=jax.ShapeDtypeStruct((N, OUTC), jnp.float32),
    mesh=_mesh,
    compiler_params=pltpu.CompilerParams(needs_layout_passes=False),
    scratch_types=[
        pltpu.VMEM((3 * PTS_W,), jnp.float32),
        pltpu.VMEM((64,), jnp.int32),
        pltpu.VMEM((64,), jnp.int32),
        pltpu.VMEM((64,), jnp.int32),
        pltpu.VMEM((64,), jnp.int32),
        pltpu.VMEM((64, 128), jnp.float32),
        pltpu.VMEM((64, 128), jnp.float32),
        pltpu.VMEM((64, 256), jnp.float32),
        pltpu.VMEM((64, 512), jnp.float32),
        pltpu.VMEM((16, 16), jnp.float32),
        pltpu.VMEM((16, OUTC), jnp.float32),
        pltpu.VMEM_SHARED((GRID,), jnp.float32),
        pltpu.VMEM((64,), jnp.int32),
        pltpu.VMEM((64,), jnp.float32),
        pltpu.VMEM((16,), jnp.int32),
        pltpu.SemaphoreType.DMA,
        pltpu.SemaphoreType.DMA,
    ],
)
def _project(coords_hbm, f0, f1, f2, f3, occ_hbm, out_hbm,
             cbuf, idx0, idx1, idx2, idx3, rows0, rows1, rows2, rows3,
             wbuf, stage, occ_sp, idxo, occv, zbuf, gsem, osem):
    cid = lax.axis_index("c")
    sid = lax.axis_index("s")
    wid = sid * NC + cid
    base = wid * PTS_W
    for a in range(3):
        pltpu.sync_copy(coords_hbm.at[pl.ds(a * NPAD + base, PTS_W)],
                        cbuf.at[pl.ds(a * PTS_W, PTS_W)])
    pltpu.sync_copy(occ_hbm.at[pl.ds(sid * (GRID // NS), GRID // NS)],
                    occ_sp.at[pl.ds(sid * (GRID // NS), GRID // NS)])
    plsc.subcore_barrier()
    nch = jnp.minimum(PTS_W, N - base) // L
    lanes = lax.iota(jnp.int32, L)
    idxs = (idx0, idx1, idx2, idx3)
    rows = (rows0, rows1, rows2, rows3)
    tables = (f0, f1, f2, f3)

    def _chunk(gi, _):
        off = gi * L
        x = cbuf[pl.ds(off, L)]
        y = cbuf[pl.ds(PTS_W + off, L)]
        z = cbuf[pl.ds(2 * PTS_W + off, L)]
        h = jnp.minimum(jnp.maximum(32.0 * y + 32.0, 0.0), 63.0)
        w = jnp.minimum(jnp.maximum(32.0 * x + 32.0, 0.0), 63.0)
        c = jnp.minimum(jnp.maximum(32.0 * z + 32.0, 0.0), 63.0)
        w0vecs = None
        for lvl in range(4):
            d = _LD[lvl]
            hx = h * _LSCALE[lvl]
            wx = w * _LSCALE[lvl]
            cx = c * _LSCALE[lvl]
            xi1 = hx.astype(jnp.int32)
            x1f = xi1.astype(jnp.float32)
            xi2 = xi1 + jnp.where(hx > x1f, 1, 0).astype(jnp.int32)
            x2f = xi2.astype(jnp.float32)
            yi1 = wx.astype(jnp.int32)
            y1f = yi1.astype(jnp.float32)
            yi2 = yi1 + jnp.where(wx > y1f, 1, 0).astype(jnp.int32)
            y2f = yi2.astype(jnp.float32)
            zt = cx.astype(jnp.int32)
            zi1 = zt + jnp.where(cx > zt.astype(jnp.float32), 1, 0).astype(jnp.int32)
            i11 = (xi1 * d + yi1) * d + zi1
            i21 = (xi2 * d + yi1) * d + zi1
            i12 = (xi1 * d + yi2) * d + zi1
            i22 = (xi2 * d + yi2) * d + zi1
            dx2 = x2f - hx
            dx1 = hx - x1f
            dy2 = y2f - wx
            dy1 = wx - y1f
            w11 = dx2 * dy2
            w21 = dx1 * dy2
            w12 = dx2 * dy1
            w22 = dx1 * dy1
            ib = idxs[lvl]
            if lvl == 0:
                ib[pl.ds(0, L)] = jnp.right_shift(i11, 1)
                ib[pl.ds(16, L)] = jnp.right_shift(i21, 1)
                ib[pl.ds(32, L)] = jnp.right_shift(i12, 1)
                ib[pl.ds(48, L)] = jnp.right_shift(i22, 1)
                idxo[pl.ds(0, L)] = i11
                idxo[pl.ds(16, L)] = i21
                idxo[pl.ds(32, L)] = i12
                idxo[pl.ds(48, L)] = i22
                zbuf[...] = (i11 & 1) * 64
            else:
                ib[pl.ds(0, L)] = i11
                ib[pl.ds(16, L)] = i21
                ib[pl.ds(32, L)] = i12
                ib[pl.ds(48, L)] = i22
            wbuf[4 * lvl + 0, :] = w11
            wbuf[4 * lvl + 1, :] = w21
            wbuf[4 * lvl + 2, :] = w12
            wbuf[4 * lvl + 3, :] = w22
            if lvl == 0:
                w0vecs = (w11, w21, w12, w22)
        dmas = [pltpu.async_copy(tables[lvl].at[idxs[lvl]], rows[lvl], gsem)
                for lvl in range(4)]
        odma = pltpu.async_copy(occ_sp.at[idxo], occv, osem)
        for dsc in dmas:
            dsc.wait()
        odma.wait()

        def _pbody(p, _):
            pfull = jnp.full((L,), p, jnp.int32)
            zv = plsc.load_gather(zbuf, [pfull])
            for lvl in range(4):
                cdim = _LCP[lvl]
                coff = _LOFF[lvl]
                rb = rows[lvl]
                wv = [plsc.load_gather(
                          wbuf,
                          [jnp.full((L,), 4 * lvl + k, jnp.int32), pfull])
                      for k in range(4)]

                def _cc(ci, _c, rb=rb, wv=wv, coff=coff, p=p,
                        pfull=pfull, lvl=lvl):
                    s = ci * L
                    if lvl == 0:
                        colv = zv + s + lanes
                        q11 = plsc.load_gather(rb, [pfull, colv])
                        q21 = plsc.load_gather(rb, [pfull + 16, colv])
                        q12 = plsc.load_gather(rb, [pfull + 32, colv])
                        q22 = plsc.load_gather(rb, [pfull + 48, colv])
                    else:
                        q11 = rb[p, pl.ds(s, L)]
                        q21 = rb[16 + p, pl.ds(s, L)]
                        q12 = rb[32 + p, pl.ds(s, L)]
                        q22 = rb[48 + p, pl.ds(s, L)]
                    plsc.store_scatter(
                        stage, [pfull, coff + s + lanes],
                        wv[0] * q11 + wv[1] * q21 + wv[2] * q12 + wv[3] * q22)
                    return 0
                lax.fori_loop(0, cdim // L, _cc, 0)
            return 0
        lax.fori_loop(0, L, _pbody, 0)

        plsc.store_scatter(stage, [lanes, jnp.full((L,), 0, jnp.int32)], x)
        plsc.store_scatter(stage, [lanes, jnp.full((L,), 1, jnp.int32)], y)
        plsc.store_scatter(stage, [lanes, jnp.full((L,), 2, jnp.int32)], z)

        acc = [jnp.zeros((L,), jnp.float32) for _ in range(3)]
        for k in range(4):
            v = occv[pl.ds(k * L, L)]
            wk = w0vecs[k]
            b2 = jnp.where(v >= 4.0, 1.0, 0.0)
            v = v - 4.0 * b2
            b1 = jnp.where(v >= 2.0, 1.0, 0.0)
            b0 = v - 2.0 * b1
            for g, bit in enumerate((b0, b1, b2)):
                acc[g] = acc[g] + wk * bit
        for g in range(3):
            for j in range(4):
                col = 951 + 4 * g + j
                plsc.store_scatter(
                    stage, [lanes, jnp.full((L,), col, jnp.int32)], acc[g])
        pltpu.sync_copy(stage, out_hbm.at[pl.ds(base + off, L)])
        return 0
    lax.fori_loop(0, nch, _chunk, 0)


def kernel(inputs, img_feat0, img_feat1, img_feat2, img_feat3,
           pc_feat0, pc_feat1, pc_feat2):
    coords = jnp.pad(inputs, ((0, NPAD - N), (0, 0)),
                     constant_values=-1.0).T.reshape(-1)
    pc = jnp.stack([pc_feat0, pc_feat1, pc_feat2], axis=0)
    pc = jnp.pad(pc, ((0, 0), (0, NPCPAD - NPC), (0, 0)))
    pc = pc.transpose(0, 2, 1).reshape(-1)
    f0 = img_feat0.reshape(GRID // 2, 128)
    f1 = img_feat1.reshape(32 * 32 * 32, 128)
    f2 = img_feat2.reshape(16 * 16 * 16, 256)
    f3 = jnp.pad(img_feat3.reshape(8 * 8 * 8, 500), ((0, 0), (0, 12)))
    occf = _occ_build(pc)
    return _project(coords, f0, f1, f2, f3, occf)


# speedup vs baseline: 11.8709x; 1.2061x over previous
import functools

import jax
import jax.numpy as jnp
from jax import lax
from jax.experimental import pallas as pl
from jax.experimental.pallas import tpu as pltpu
from jax.experimental.pallas import tpu_sc as plsc

NC, NS, L = 2, 16, 16
NW = NC * NS
N = 100000
PTS_W = 3136
NPAD = NW * PTS_W
NCH = PTS_W // L
NPC = 50000
PCW = 3136
NPCPAD = NS * PCW
GRID = 64 * 64 * 64
OUTC = 963

_LD = (64, 32, 16, 8)
_LC = (64, 128, 256, 500)
_LCP = (64, 128, 256, 512)
_LOFF = (3, 67, 195, 451)
_LSCALE = (1.0, 0.5, 0.25, 0.125)

_mesh = plsc.VectorSubcoreMesh(
    core_axis_name="c", subcore_axis_name="s", num_cores=NC, num_subcores=NS)


@functools.partial(
    pl.kernel,
    out_type=jax.ShapeDtypeStruct((GRID,), jnp.float32),
    mesh=_mesh,
    compiler_params=pltpu.CompilerParams(needs_layout_passes=False),
    scratch_types=[
        pltpu.VMEM((PCW,), jnp.float32),
        pltpu.VMEM((PCW,), jnp.float32),
        pltpu.VMEM((PCW,), jnp.float32),
        pltpu.VMEM((112,), jnp.int32),
        pltpu.VMEM((112,), jnp.float32),
        pltpu.VMEM_SHARED((3 * GRID,), jnp.float32),
        pltpu.VMEM((3 * 8192,), jnp.float32),
        pltpu.VMEM((8192,), jnp.float32),
    ],
)
def _occ_build(pc_hbm, occ_hbm, px, py, pz, idxb, valb, grids, gbuf, obuf):
    cid = lax.axis_index("c")
    sid = lax.axis_index("s")
    wid = sid * NC + cid
    lanes = lax.iota(jnp.int32, L)
    zeros16 = jnp.zeros((L,), jnp.float32)

    def _zb(i, _):
        gbuf[pl.ds(i * L, L)] = zeros16
        return 0
    lax.fori_loop(0, 8192 // L, _zb, 0)
    for j in range(6):
        pltpu.sync_copy(gbuf.at[pl.ds(0, 8192)],
                        grids.at[pl.ds(sid * (3 * GRID // NS) + j * 8192, 8192)])
    plsc.subcore_barrier()

    for g in range(3):
        gb = 3 * g * NPCPAD
        pltpu.sync_copy(pc_hbm.at[pl.ds(gb + sid * PCW, PCW)], px)
        pltpu.sync_copy(pc_hbm.at[pl.ds(gb + NPCPAD + sid * PCW, PCW)], py)
        pltpu.sync_copy(pc_hbm.at[pl.ds(gb + 2 * NPCPAD + sid * PCW, PCW)], pz)

        def _grp(j, _, g=g):
            for t in range(7):
                o = t * L
                off = j * 112 + o
                x = px[pl.ds(off, L)]
                y = py[pl.ds(off, L)]
                z = pz[pl.ds(off, L)]
                h1 = jnp.minimum(jnp.maximum(32.0 * y + 32.0, 0.0),
                                 63.0).astype(jnp.int32)
                w1 = jnp.minimum(jnp.maximum(32.0 * x + 32.0, 0.0),
                                 63.0).astype(jnp.int32)
                c1 = jnp.minimum(jnp.maximum(32.0 * z + 32.0, 0.0),
                                 63.0).astype(jnp.int32)
                flat = (h1 * 64 + w1) * 64 + c1 + g * GRID
                gidx = sid * PCW + off + lanes
                val = jnp.where(gidx < NPC, 1.0, 0.0).astype(jnp.float32)
                idxb[pl.ds(o, L)] = flat
                valb[pl.ds(o, L)] = val
            pltpu.sync_copy(valb, grids.at[idxb], add=True)
            return 0
        lax.fori_loop(0, PCW // 112, _grp, 0)
    plsc.subcore_barrier()

    base = wid * (GRID // NW)
    for g in range(3):
        pltpu.sync_copy(grids.at[pl.ds(g * GRID + base, 8192)],
                        gbuf.at[pl.ds(g * 8192, 8192)])

    def _pk(i, _):
        s = i * L
        b0 = jnp.where(gbuf[pl.ds(s, L)] > 0.5, 1.0, 0.0)
        b1 = jnp.where(gbuf[pl.ds(8192 + s, L)] > 0.5, 2.0, 0.0)
        b2 = jnp.where(gbuf[pl.ds(16384 + s, L)] > 0.5, 4.0, 0.0)
        obuf[pl.ds(s, L)] = (b0 + b1 + b2).astype(jnp.float32)
        return 0
    lax.fori_loop(0, 8192 // L, _pk, 0)
    pltpu.sync_copy(obuf, occ_hbm.at[pl.ds(base, 8192)])


@functools.partial(
    pl.kernel,
    out_type=jax.ShapeDtypeStruct((N, OUTC), jnp.float32),
    mesh=_mesh,
    compiler_params=pltpu.CompilerParams(needs_layout_passes=False),
    scratch_types=[
        pltpu.VMEM((3 * PTS_W + 16,), jnp.float32),
        pltpu.VMEM((64,), jnp.int32),
        pltpu.VMEM((64,), jnp.int32),
        pltpu.VMEM((64,), jnp.int32),
        pltpu.VMEM((64,), jnp.int32),
        pltpu.VMEM((64,), jnp.int32),
        pltpu.VMEM((64,), jnp.int32),
        pltpu.VMEM((64,), jnp.int32),
        pltpu.VMEM((64,), jnp.int32),
        pltpu.VMEM((64,), jnp.int32),
        pltpu.VMEM((64,), jnp.int32),
        pltpu.VMEM((64, 128), jnp.float32),
        pltpu.VMEM((64, 128), jnp.float32),
        pltpu.VMEM((64, 256), jnp.float32),
        pltpu.VMEM((64, 512), jnp.float32),
        pltpu.VMEM((256,), jnp.float32),
        pltpu.VMEM((256,), jnp.float32),
        pltpu.VMEM((16,), jnp.int32),
        pltpu.VMEM((16,), jnp.int32),
        pltpu.VMEM((16, OUTC), jnp.float32),
        pltpu.VMEM((16, OUTC), jnp.float32),
        pltpu.VMEM_SHARED((GRID,), jnp.float32),
        pltpu.VMEM((64,), jnp.float32),
        pltpu.SemaphoreType.DMA,
        pltpu.SemaphoreType.DMA,
        pltpu.SemaphoreType.DMA,
    ],
)
def _project(coords_hbm, f0, f1, f2, f3, occ_hbm, out_hbm,
             cbuf, idx0a, idx1a, idx2a, idx3a, idxoa,
             idx0b, idx1b, idx2b, idx3b, idxob,
             rows0, rows1, rows2, rows3, wbufa, wbufb, zbufa, zbufb,
             stagea, stageb, occ_sp, occv, gsem, osem, wsem):
    cid = lax.axis_index("c")
    sid = lax.axis_index("s")
    wid = sid * NC + cid
    base = wid * PTS_W
    for a in range(3):
        pltpu.sync_copy(coords_hbm.at[pl.ds(a * NPAD + base, PTS_W)],
                        cbuf.at[pl.ds(a * PTS_W, PTS_W)])
    pltpu.sync_copy(occ_hbm.at[pl.ds(sid * (GRID // NS), GRID // NS)],
                    occ_sp.at[pl.ds(sid * (GRID // NS), GRID // NS)])
    plsc.subcore_barrier()
    nch = jnp.minimum(PTS_W, N - base) // L
    lanes = lax.iota(jnp.int32, L)
    rows = (rows0, rows1, rows2, rows3)
    tables = (f0, f1, f2, f3)
    seta = (idx0a, idx1a, idx2a, idx3a, idxoa, wbufa, zbufa, stagea)
    setb = (idx0b, idx1b, idx2b, idx3b, idxob, wbufb, zbufb, stageb)

    def _idxcomp(ci, st):
        i0, i1, i2, i3, io, wb, zb, _ = st
        idxs = (i0, i1, i2, i3)
        off = ci * L
        x = cbuf[pl.ds(off, L)]
        y = cbuf[pl.ds(PTS_W + off, L)]
        z = cbuf[pl.ds(2 * PTS_W + off, L)]
        h = jnp.minimum(jnp.maximum(32.0 * y + 32.0, 0.0), 63.0)
        w = jnp.minimum(jnp.maximum(32.0 * x + 32.0, 0.0), 63.0)
        c = jnp.minimum(jnp.maximum(32.0 * z + 32.0, 0.0), 63.0)
        for lvl in range(4):
            d = _LD[lvl]
            hx = h * _LSCALE[lvl]
            wx = w * _LSCALE[lvl]
            cx = c * _LSCALE[lvl]
            xi1 = hx.astype(jnp.int32)
            x1f = xi1.astype(jnp.float32)
            xi2 = xi1 + jnp.where(hx > x1f, 1, 0).astype(jnp.int32)
            x2f = xi2.astype(jnp.float32)
            yi1 = wx.astype(jnp.int32)
            y1f = yi1.astype(jnp.float32)
            yi2 = yi1 + jnp.where(wx > y1f, 1, 0).astype(jnp.int32)
            y2f = yi2.astype(jnp.float32)
            zt = cx.astype(jnp.int32)
            zi1 = zt + jnp.where(cx > zt.astype(jnp.float32), 1, 0).astype(jnp.int32)
            i11 = (xi1 * d + yi1) * d + zi1
            i21 = (xi2 * d + yi1) * d + zi1
            i12 = (xi1 * d + yi2) * d + zi1
            i22 = (xi2 * d + yi2) * d + zi1
            dx2 = x2f - hx
            dx1 = hx - x1f
            dy2 = y2f - wx
            dy1 = wx - y1f
            ib = idxs[lvl]
            if lvl == 0:
                ib[pl.ds(0, L)] = jnp.right_shift(i11, 1)
                ib[pl.ds(16, L)] = jnp.right_shift(i21, 1)
                ib[pl.ds(32, L)] = jnp.right_shift(i12, 1)
                ib[pl.ds(48, L)] = jnp.right_shift(i22, 1)
                io[pl.ds(0, L)] = i11
                io[pl.ds(16, L)] = i21
                io[pl.ds(32, L)] = i12
                io[pl.ds(48, L)] = i22
                zb[...] = (i11 & 1) * 64
            else:
                ib[pl.ds(0, L)] = i11
                ib[pl.ds(16, L)] = i21
                ib[pl.ds(32, L)] = i12
                ib[pl.ds(48, L)] = i22
            wb[pl.ds(64 * lvl + 0, L)] = dx2 * dy2
            wb[pl.ds(64 * lvl + 16, L)] = dx1 * dy2
            wb[pl.ds(64 * lvl + 32, L)] = dx2 * dy1
            wb[pl.ds(64 * lvl + 48, L)] = dx1 * dy1

    def _fire(st):
        for lvl in range(4):
            pltpu.async_copy(tables[lvl].at[st[lvl]], rows[lvl], gsem)
        pltpu.async_copy(occ_sp.at[st[4]], occv, osem)

    def _wait_gathers():
        for lvl in range(4):
            pltpu.make_async_copy(
                tables[lvl].at[pl.ds(0, 64)], rows[lvl], gsem).wait()
        pltpu.make_async_copy(occ_hbm.at[pl.ds(0, 64)], occv, osem).wait()

    def _combine_lvl(lvl, st):
        wb, zb, stage = st[5], st[6], st[7]
        cdim = _LCP[lvl]
        coff = _LOFF[lvl]
        rb = rows[lvl]

        def _pb(p, _):
            pfull = jnp.full((L,), p, jnp.int32)
            wv = [plsc.load_gather(wb, [64 * lvl + k * 16 + pfull])
                  for k in range(4)]
            if lvl == 0:
                zv = plsc.load_gather(zb, [pfull])

            def _cc(ci, _c):
                s = ci * L
                if lvl == 0:
                    colv = zv + s + lanes
                    q11 = plsc.load_gather(rb, [pfull, colv])
                    q21 = plsc.load_gather(rb, [pfull + 16, colv])
                    q12 = plsc.load_gather(rb, [pfull + 32, colv])
                    q22 = plsc.load_gather(rb, [pfull + 48, colv])
                else:
                    q11 = rb[p, pl.ds(s, L)]
                    q21 = rb[16 + p, pl.ds(s, L)]
                    q12 = rb[32 + p, pl.ds(s, L)]
                    q22 = rb[48 + p, pl.ds(s, L)]
                plsc.store_scatter(
                    stage, [pfull, coff + s + lanes],
                    wv[0] * q11 + wv[1] * q21 + wv[2] * q12 + wv[3] * q22)
                return 0
            lax.fori_loop(0, cdim // L, _cc, 0)
            return 0
        lax.fori_loop(0, L, _pb, 0)

    def _combine_occ_coords(ci, st):
        wb, stage = st[5], st[7]
        off = ci * L
        x = cbuf[pl.ds(off, L)]
        y = cbuf[pl.ds(PTS_W + off, L)]
        z = cbuf[pl.ds(2 * PTS_W + off, L)]
        plsc.store_scatter(stage, [lanes, jnp.full((L,), 0, jnp.int32)], x)
        plsc.store_scatter(stage, [lanes, jnp.full((L,), 1, jnp.int32)], y)
        plsc.store_scatter(stage, [lanes, jnp.full((L,), 2, jnp.int32)], z)
        acc = [jnp.zeros((L,), jnp.float32) for _ in range(3)]
        for k in range(4):
            v = occv[pl.ds(k * L, L)]
            wk = wb[pl.ds(k * 16, L)]
            b2 = jnp.where(v >= 4.0, 1.0, 0.0)
            v = v - 4.0 * b2
            b1 = jnp.where(v >= 2.0, 1.0, 0.0)
            b0 = v - 2.0 * b1
            for g, bit in enumerate((b0, b1, b2)):
                acc[g] = acc[g] + wk * bit
        for g in range(3):
            for j in range(4):
                col = 951 + 4 * g + j
                plsc.store_scatter(
                    stage, [lanes, jnp.full((L,), col, jnp.int32)], acc[g])

    _idxcomp(0, seta)
    _fire(seta)

    def _half(ci2, ci, cur, nxt):
        @pl.when(ci2 >= 1)
        def _():
            pltpu.make_async_copy(out_hbm.at[pl.ds(0, L)], cur[7], wsem).wait()
        _wait_gathers()
        _idxcomp(ci + 1, nxt)
        for lvl in (3, 2, 1):
            _combine_lvl(lvl, cur)

            @pl.when(ci < nch - 1)
            def _(lvl=lvl):
                pltpu.async_copy(tables[lvl].at[nxt[lvl]], rows[lvl], gsem)
        _combine_lvl(0, cur)
        _combine_occ_coords(ci, cur)

        @pl.when(ci < nch - 1)
        def _():
            pltpu.async_copy(tables[0].at[nxt[0]], rows[0], gsem)
            pltpu.async_copy(occ_sp.at[nxt[4]], occv, osem)
        pltpu.async_copy(cur[7], out_hbm.at[pl.ds(base + ci * L, L)], wsem)

    def _body(ci2, _):
        _half(ci2, 2 * ci2, seta, setb)
        _half(ci2, 2 * ci2 + 1, setb, seta)
        return 0
    lax.fori_loop(0, nch // 2, _body, 0)
    pltpu.make_async_copy(out_hbm.at[pl.ds(0, L)], stagea, wsem).wait()
    pltpu.make_async_copy(out_hbm.at[pl.ds(0, L)], stageb, wsem).wait()


def kernel(inputs, img_feat0, img_feat1, img_feat2, img_feat3,
           pc_feat0, pc_feat1, pc_feat2):
    coords = jnp.pad(inputs, ((0, NPAD - N), (0, 0)),
                     constant_values=-1.0).T.reshape(-1)
    pc = jnp.stack([pc_feat0, pc_feat1, pc_feat2], axis=0)
    pc = jnp.pad(pc, ((0, 0), (0, NPCPAD - NPC), (0, 0)))
    pc = pc.transpose(0, 2, 1).reshape(-1)
    f0 = img_feat0.reshape(GRID // 2, 128)
    f1 = img_feat1.reshape(32 * 32 * 32, 128)
    f2 = img_feat2.reshape(16 * 16 * 16, 256)
    f3 = jnp.pad(img_feat3.reshape(8 * 8 * 8, 500), ((0, 0), (0, 12)))
    occf = _occ_build(pc)
    return _project(coords, f0, f1, f2, f3, occf)
